# Initial kernel scaffold; baseline (speedup 1.0000x reference)
import jax
import jax.numpy as jnp
from jax.experimental import pallas as pl


def kernel(b, N, K, L, att_feats, obj_dist, pred_fmap, rel_ind, W_obj, W_ps, W_po, W_pred, W_sp, W_op):
    raise NotImplementedError("write your pallas kernel here")


# speedup vs baseline: 5.5647x; 5.5647x over previous
import jax
import jax.numpy as jnp
from jax.experimental import pallas as pl

LAYERS = 2
RES = 2


def _gcn_body(att_ref, pred_ref, ind_ref,
              w_obj_ref, w_ps_ref, w_po_ref, w_pred_ref, w_sp_ref, w_op_ref,
              out_obj_ref, out_pred_ref):
    x_obj = att_ref[0]
    x_pred = pred_ref[0]
    ind_s = ind_ref[0, :, 0]
    ind_o = ind_ref[0, :, 1]

    n = x_obj.shape[0]
    k = x_pred.shape[0]
    iota_n = jax.lax.broadcasted_iota(jnp.int32, (k, n), 1)
    map_sT = (ind_s[:, None] == iota_n).astype(jnp.float32)
    map_oT = (ind_o[:, None] == iota_n).astype(jnp.float32)

    res_obj = x_obj
    res_pred = x_pred
    dn = (((0,), (0,)), ((), ()))
    for i in range(LAYERS):
        agg_obj_s = jax.lax.dot_general(map_sT, x_pred, dn,
                                        preferred_element_type=jnp.float32)
        agg_obj_o = jax.lax.dot_general(map_oT, x_pred, dn,
                                        preferred_element_type=jnp.float32)
        agg_pred_s = jnp.dot(map_sT, x_obj, preferred_element_type=jnp.float32)
        agg_pred_o = jnp.dot(map_oT, x_obj, preferred_element_type=jnp.float32)
        new_obj = jax.nn.relu(
            jnp.dot(x_obj, w_obj_ref[i], preferred_element_type=jnp.float32)
            + jnp.dot(agg_obj_s, w_ps_ref[i], preferred_element_type=jnp.float32)
            + jnp.dot(agg_obj_o, w_po_ref[i], preferred_element_type=jnp.float32))
        new_pred = jax.nn.relu(
            jnp.dot(x_pred, w_pred_ref[i], preferred_element_type=jnp.float32)
            + jnp.dot(agg_pred_s, w_sp_ref[i], preferred_element_type=jnp.float32)
            + jnp.dot(agg_pred_o, w_op_ref[i], preferred_element_type=jnp.float32))
        x_obj, x_pred = new_obj, new_pred
        if (i + 1) % RES == 0:
            x_obj = x_obj + res_obj
            res_obj = x_obj
            x_pred = x_pred + res_pred
            res_pred = x_pred

    for c in range(5):
        out_obj_ref[0, c] = x_obj
        out_pred_ref[0, c] = x_pred


def kernel(b, N, K, L, att_feats, obj_dist, pred_fmap, rel_ind,
           W_obj, W_ps, W_po, W_pred, W_sp, W_op):
    del obj_dist, b, N, K, L
    b, N, L = att_feats.shape
    K = pred_fmap.shape[1]
    grid = (b,)
    out_obj, out_pred = pl.pallas_call(
        _gcn_body,
        grid=grid,
        in_specs=[
            pl.BlockSpec((1, N, L), lambda i: (i, 0, 0)),
            pl.BlockSpec((1, K, L), lambda i: (i, 0, 0)),
            pl.BlockSpec((1, K, 2), lambda i: (i, 0, 0)),
            pl.BlockSpec((LAYERS, L, L), lambda i: (0, 0, 0)),
            pl.BlockSpec((LAYERS, L, L), lambda i: (0, 0, 0)),
            pl.BlockSpec((LAYERS, L, L), lambda i: (0, 0, 0)),
            pl.BlockSpec((LAYERS, L, L), lambda i: (0, 0, 0)),
            pl.BlockSpec((LAYERS, L, L), lambda i: (0, 0, 0)),
            pl.BlockSpec((LAYERS, L, L), lambda i: (0, 0, 0)),
        ],
        out_specs=[
            pl.BlockSpec((1, 5, N, L), lambda i: (i, 0, 0, 0)),
            pl.BlockSpec((1, 5, K, L), lambda i: (i, 0, 0, 0)),
        ],
        out_shape=[
            jax.ShapeDtypeStruct((b, 5, N, L), jnp.float32),
            jax.ShapeDtypeStruct((b, 5, K, L), jnp.float32),
        ],
    )(att_feats, pred_fmap, rel_ind, W_obj, W_ps, W_po, W_pred, W_sp, W_op)
    return (out_obj.reshape(b * 5, N, L), out_pred.reshape(b * 5, K, L))
